# baseline (speedup 1.0000x reference)
import math
import jax
import jax.numpy as jnp
from jax import lax
from jax.experimental import pallas as pl
from jax.experimental.pallas import tpu as pltpu
from jax.experimental.pallas import tpu_sc as plsc

D = 128
SCALE = math.sqrt(D)
W = 128
N_WORKERS = 32


def _seg_embedding_sc(xi, pi, si, word_emb, pos_emb, seg_emb):
    n_tok = xi.shape[0]
    per_w = n_tok // N_WORKERS
    n_chunks = per_w // W
    max_len = pos_emb.shape[0]
    max_seg = seg_emb.shape[0]
    mesh = plsc.VectorSubcoreMesh(core_axis_name="core",
                                  subcore_axis_name="subcore")

    @pl.kernel(
        out_type=jax.ShapeDtypeStruct((n_tok, D), jnp.float32),
        mesh=mesh,
        scratch_types=[
            pltpu.VMEM_SHARED((max_len, D), jnp.float32),
            pltpu.VMEM_SHARED((max_seg, D), jnp.float32),
            pltpu.VMEM((per_w,), jnp.int32),
            pltpu.VMEM((per_w,), jnp.int32),
            pltpu.VMEM((W,), jnp.int32),
            pltpu.VMEM((W,), jnp.int32),
            pltpu.VMEM((W, D), jnp.float32),
            pltpu.VMEM((W, D), jnp.float32),
            pltpu.VMEM((W, D), jnp.float32),
            pltpu.VMEM((W, D), jnp.float32),
            pltpu.SemaphoreType.DMA,
            pltpu.SemaphoreType.DMA,
            pltpu.SemaphoreType.DMA,
            pltpu.SemaphoreType.DMA,
            pltpu.SemaphoreType.DMA,
            pltpu.SemaphoreType.DMA,
            pltpu.SemaphoreType.DMA,
            pltpu.SemaphoreType.DMA,
            pltpu.SemaphoreType.DMA,
            pltpu.SemaphoreType.DMA,
        ],
    )
    def kern(word_hbm, pos_hbm, seg_hbm, xi_hbm, pi_hbm, si_hbm, o_hbm,
             pos_sh, seg_sh, pv, sv, ix0, ix1, w0, w1, ps0, ps1,
             six0, six1, sw0, sw1, sp0, sp1, ss0, ss1, so0, so1):
        sid = lax.axis_index("subcore")
        wid = lax.axis_index("core") * 16 + sid
        base = wid * per_w
        ixbuf = (ix0, ix1)
        wbuf = (w0, w1)
        psbuf = (ps0, ps1)
        six = (six0, six1)
        sw = (sw0, sw1)
        sp = (sp0, sp1)
        ss = (ss0, ss1)
        so = (so0, so1)

        pr = (max_len // 16) // 8 * 8
        sr = (max_seg // 16) // 8 * 8
        pltpu.sync_copy(pos_hbm.at[pl.ds(sid * pr, pr)],
                        pos_sh.at[pl.ds(sid * pr, pr)])
        pltpu.sync_copy(seg_hbm.at[pl.ds(sid * sr, sr)],
                        seg_sh.at[pl.ds(sid * sr, sr)])

        prem = max_len - 16 * pr
        srem = max_seg - 16 * sr
        if prem:
            @pl.when(sid == 0)
            def _():
                pltpu.sync_copy(pos_hbm.at[pl.ds(16 * pr, prem)],
                                pos_sh.at[pl.ds(16 * pr, prem)])
        if srem:
            @pl.when(sid == 1)
            def _():
                pltpu.sync_copy(seg_hbm.at[pl.ds(16 * sr, srem)],
                                seg_sh.at[pl.ds(16 * sr, srem)])

        plsc.subcore_barrier()

        cp_ = pltpu.async_copy(pi_hbm.at[pl.ds(base, per_w)], pv, sp0)
        cs_ = pltpu.async_copy(si_hbm.at[pl.ds(base, per_w)], sv, ss0)
        cp_.wait()
        cs_.wait()

        def load_ix(c, q):
            pltpu.async_copy(xi_hbm.at[pl.ds(base + c * W, W)],
                             ixbuf[q], six[q])

        def issue_w(q):
            pltpu.async_copy(word_hbm.at[ixbuf[q]], wbuf[q], sw[q])

        def issue_p(c, q):
            pltpu.async_copy(pos_sh.at[pv.at[pl.ds(c * W, W)]],
                             psbuf[q], sp[q])

        def issue_s(c, q):
            pltpu.async_copy(seg_sh.at[sv.at[pl.ds(c * W, W)]],
                             psbuf[q], ss[q], add=True)

        def wait_rows(sem, buf):
            pltpu.make_async_copy(o_hbm.at[pl.ds(0, W)], buf, sem).wait()

        def wait_ix(q):
            pltpu.make_async_copy(xi_hbm.at[pl.ds(0, W)], ixbuf[q],
                                  six[q]).wait()

        load_ix(0, 0)
        load_ix(1, 1)
        wait_ix(0)
        issue_w(0)
        issue_p(0, 0)
        wait_rows(sp[0], psbuf[0])
        issue_s(0, 0)

        def body(c, q):
            wait_rows(sw[q], wbuf[q])

            @pl.when(c + 2 < n_chunks)
            def _():
                load_ix(c + 2, q)

            @pl.when(c + 1 < n_chunks)
            def _():
                wait_ix(1 - q)
                issue_w(1 - q)

            wait_rows(ss[q], psbuf[q])

            @pl.when(c + 1 < n_chunks)
            def _():
                @pl.when(c >= 1)
                def _():
                    pltpu.make_async_copy(
                        psbuf[1 - q], o_hbm.at[pl.ds(base, W)],
                        so[1 - q]).wait()

                issue_p(c + 1, 1 - q)

            @plsc.parallel_loop(0, W, step=1, unroll=4)
            def _(r):
                for col in range(0, D, 16):
                    sl = (r, pl.ds(col, 16))
                    psbuf[q][sl] = wbuf[q][sl] * SCALE + psbuf[q][sl]

            @pl.when(c + 1 < n_chunks)
            def _():
                wait_rows(sp[1 - q], psbuf[1 - q])
                issue_s(c + 1, 1 - q)

            pltpu.async_copy(psbuf[q], o_hbm.at[pl.ds(base + c * W, W)],
                             so[q])

        @pl.loop(0, n_chunks, step=2)
        def _(c):
            body(c, 0)
            body(c + 1, 1)

        pltpu.make_async_copy(psbuf[0], o_hbm.at[pl.ds(base, W)], so[0]).wait()
        pltpu.make_async_copy(psbuf[1], o_hbm.at[pl.ds(base, W)], so[1]).wait()

    return kern(word_emb, pos_emb, seg_emb, xi, pi, si)


def kernel(x, pos, seg, word_emb, pos_emb, seg_emb):
    b, l = x.shape
    n_tok = b * l
    xi = x.reshape(n_tok).astype(jnp.int32)
    pi = pos.reshape(n_tok).astype(jnp.int32)
    si = seg.reshape(n_tok).astype(jnp.int32)
    out = _seg_embedding_sc(xi, pi, si, word_emb, pos_emb, seg_emb)
    return out.reshape(b, l, D)


# speedup vs baseline: 1.0046x; 1.0046x over previous
import math
import jax
import jax.numpy as jnp
from jax import lax
from jax.experimental import pallas as pl
from jax.experimental.pallas import tpu as pltpu
from jax.experimental.pallas import tpu_sc as plsc

D = 128
SCALE = math.sqrt(D)
W = 128
N_WORKERS = 32


def _seg_embedding_sc(xi, pi, si, word_emb, pos_emb, seg_emb):
    n_tok = xi.shape[0]
    per_w = n_tok // N_WORKERS
    n_chunks = per_w // W
    max_len = pos_emb.shape[0]
    max_seg = seg_emb.shape[0]
    mesh = plsc.VectorSubcoreMesh(core_axis_name="core",
                                  subcore_axis_name="subcore")

    @pl.kernel(
        out_type=jax.ShapeDtypeStruct((n_tok, D), jnp.float32),
        mesh=mesh,
        scratch_types=[
            pltpu.VMEM_SHARED((max_len, D), jnp.float32),
            pltpu.VMEM_SHARED((max_seg, D), jnp.float32),
            pltpu.VMEM((per_w,), jnp.int32),
            pltpu.VMEM((per_w,), jnp.int32),
            pltpu.VMEM((W,), jnp.int32),
            pltpu.VMEM((W,), jnp.int32),
            pltpu.VMEM((W, D), jnp.float32),
            pltpu.VMEM((W, D), jnp.float32),
            pltpu.VMEM((W, D), jnp.float32),
            pltpu.VMEM((W, D), jnp.float32),
            pltpu.SemaphoreType.DMA,
            pltpu.SemaphoreType.DMA,
            pltpu.SemaphoreType.DMA,
            pltpu.SemaphoreType.DMA,
            pltpu.SemaphoreType.DMA,
            pltpu.SemaphoreType.DMA,
            pltpu.SemaphoreType.DMA,
            pltpu.SemaphoreType.DMA,
            pltpu.SemaphoreType.DMA,
            pltpu.SemaphoreType.DMA,
        ],
    )
    def kern(word_hbm, pos_hbm, seg_hbm, xi_hbm, pi_hbm, si_hbm, o_hbm,
             pos_sh, seg_sh, pv, sv, ix0, ix1, w0, w1, ps0, ps1,
             six0, six1, sw0, sw1, sp0, sp1, ss0, ss1, so0, so1):
        sid = lax.axis_index("subcore")
        wid = lax.axis_index("core") * 16 + sid
        base = wid * per_w
        ixbuf = (ix0, ix1)
        wbuf = (w0, w1)
        psbuf = (ps0, ps1)
        six = (six0, six1)
        sw = (sw0, sw1)
        sp = (sp0, sp1)
        ss = (ss0, ss1)
        so = (so0, so1)

        pr = (max_len // 16) // 8 * 8
        sr = (max_seg // 16) // 8 * 8
        pltpu.sync_copy(pos_hbm.at[pl.ds(sid * pr, pr)],
                        pos_sh.at[pl.ds(sid * pr, pr)])
        pltpu.sync_copy(seg_hbm.at[pl.ds(sid * sr, sr)],
                        seg_sh.at[pl.ds(sid * sr, sr)])

        prem = max_len - 16 * pr
        srem = max_seg - 16 * sr
        if prem:
            @pl.when(sid == 0)
            def _():
                pltpu.sync_copy(pos_hbm.at[pl.ds(16 * pr, prem)],
                                pos_sh.at[pl.ds(16 * pr, prem)])
        if srem:
            @pl.when(sid == 1)
            def _():
                pltpu.sync_copy(seg_hbm.at[pl.ds(16 * sr, srem)],
                                seg_sh.at[pl.ds(16 * sr, srem)])

        plsc.subcore_barrier()

        cp_ = pltpu.async_copy(pi_hbm.at[pl.ds(base, per_w)], pv, sp0)
        cs_ = pltpu.async_copy(si_hbm.at[pl.ds(base, per_w)], sv, ss0)
        cp_.wait()
        cs_.wait()

        def load_ix(c, q):
            pltpu.async_copy(xi_hbm.at[pl.ds(base + c * W, W)],
                             ixbuf[q], six[q])

        def issue_w(q):
            pltpu.async_copy(word_hbm.at[ixbuf[q]], wbuf[q], sw[q])

        def issue_p(c, q):
            pltpu.async_copy(pos_sh.at[pv.at[pl.ds(c * W, W)]],
                             psbuf[q], sp[q])

        def issue_s(c, q):
            pltpu.async_copy(seg_sh.at[sv.at[pl.ds(c * W, W)]],
                             psbuf[q], ss[q], add=True)

        def wait_rows(sem, buf):
            pltpu.make_async_copy(o_hbm.at[pl.ds(0, W)], buf, sem).wait()

        def wait_ix(q):
            pltpu.make_async_copy(xi_hbm.at[pl.ds(0, W)], ixbuf[q],
                                  six[q]).wait()

        load_ix(0, 0)
        load_ix(1, 1)
        wait_ix(0)
        issue_w(0)
        issue_p(0, 0)
        wait_rows(sp[0], psbuf[0])
        issue_s(0, 0)

        def body(c, q):
            @pl.when(c + 1 < n_chunks)
            def _():
                wait_ix(1 - q)
                issue_w(1 - q)

            wait_rows(sw[q], wbuf[q])

            @pl.when(c + 2 < n_chunks)
            def _():
                load_ix(c + 2, q)

            wait_rows(ss[q], psbuf[q])

            @pl.when(c + 1 < n_chunks)
            def _():
                @pl.when(c >= 1)
                def _():
                    pltpu.make_async_copy(
                        psbuf[1 - q], o_hbm.at[pl.ds(base, W)],
                        so[1 - q]).wait()

                issue_p(c + 1, 1 - q)

            @plsc.parallel_loop(0, W, step=1, unroll=4)
            def _(r):
                for col in range(0, D, 16):
                    sl = (r, pl.ds(col, 16))
                    psbuf[q][sl] = wbuf[q][sl] * SCALE + psbuf[q][sl]

            pltpu.async_copy(psbuf[q], o_hbm.at[pl.ds(base + c * W, W)],
                             so[q])

            @pl.when(c + 1 < n_chunks)
            def _():
                wait_rows(sp[1 - q], psbuf[1 - q])
                issue_s(c + 1, 1 - q)

        @pl.loop(0, n_chunks, step=2)
        def _(c):
            body(c, 0)
            body(c + 1, 1)

        pltpu.make_async_copy(psbuf[0], o_hbm.at[pl.ds(base, W)], so[0]).wait()
        pltpu.make_async_copy(psbuf[1], o_hbm.at[pl.ds(base, W)], so[1]).wait()

    return kern(word_emb, pos_emb, seg_emb, xi, pi, si)


def kernel(x, pos, seg, word_emb, pos_emb, seg_emb):
    b, l = x.shape
    n_tok = b * l
    xi = x.reshape(n_tok).astype(jnp.int32)
    pi = pos.reshape(n_tok).astype(jnp.int32)
    si = seg.reshape(n_tok).astype(jnp.int32)
    out = _seg_embedding_sc(xi, pi, si, word_emb, pos_emb, seg_emb)
    return out.reshape(b, l, D)


# speedup vs baseline: 1.0072x; 1.0026x over previous
import math
import jax
import jax.numpy as jnp
from jax import lax
from jax.experimental import pallas as pl
from jax.experimental.pallas import tpu as pltpu
from jax.experimental.pallas import tpu_sc as plsc

D = 128
SCALE = math.sqrt(D)
W = 128
N_WORKERS = 32


def _seg_embedding_sc(xi, pi, si, word_emb, pos_emb, seg_emb):
    n_tok = xi.shape[0]
    per_w = n_tok // N_WORKERS
    n_chunks = per_w // W
    max_len = pos_emb.shape[0]
    max_seg = seg_emb.shape[0]
    mesh = plsc.VectorSubcoreMesh(core_axis_name="core",
                                  subcore_axis_name="subcore")

    @pl.kernel(
        out_type=jax.ShapeDtypeStruct((n_tok, D), jnp.float32),
        mesh=mesh,
        scratch_types=[
            pltpu.VMEM_SHARED((max_len, D), jnp.float32),
            pltpu.VMEM_SHARED((max_seg, D), jnp.float32),
            pltpu.VMEM((per_w,), jnp.int32),
            pltpu.VMEM((per_w,), jnp.int32),
            pltpu.VMEM((W,), jnp.int32),
            pltpu.VMEM((W,), jnp.int32),
            pltpu.VMEM((W, D), jnp.float32),
            pltpu.VMEM((W, D), jnp.float32),
            pltpu.VMEM((W, D), jnp.float32),
            pltpu.VMEM((W, D), jnp.float32),
            pltpu.SemaphoreType.DMA,
            pltpu.SemaphoreType.DMA,
            pltpu.SemaphoreType.DMA,
            pltpu.SemaphoreType.DMA,
            pltpu.SemaphoreType.DMA,
            pltpu.SemaphoreType.DMA,
            pltpu.SemaphoreType.DMA,
            pltpu.SemaphoreType.DMA,
            pltpu.SemaphoreType.DMA,
            pltpu.SemaphoreType.DMA,
        ],
    )
    def kern(word_hbm, pos_hbm, seg_hbm, xi_hbm, pi_hbm, si_hbm, o_hbm,
             pos_sh, seg_sh, pv, sv, ix0, ix1, w0, w1, ps0, ps1,
             six0, six1, sw0, sw1, sp0, sp1, ss0, ss1, so0, so1):
        sid = lax.axis_index("subcore")
        wid = lax.axis_index("core") * 16 + sid
        base = wid * per_w
        ixbuf = (ix0, ix1)
        wbuf = (w0, w1)
        psbuf = (ps0, ps1)
        six = (six0, six1)
        sw = (sw0, sw1)
        sp = (sp0, sp1)
        ss = (ss0, ss1)
        so = (so0, so1)

        c_ix0 = pltpu.async_copy(xi_hbm.at[pl.ds(base, W)], ix0, six0)
        c_ix1 = pltpu.async_copy(xi_hbm.at[pl.ds(base + W, W)], ix1, six1)
        c_pv = pltpu.async_copy(pi_hbm.at[pl.ds(base, per_w)], pv, sp0)
        c_sv = pltpu.async_copy(si_hbm.at[pl.ds(base, per_w)], sv, ss0)
        c_ix0.wait()
        pltpu.async_copy(word_hbm.at[ix0], w0, sw0)

        pr = (max_len // 16) // 8 * 8
        sr = (max_seg // 16) // 8 * 8
        pltpu.sync_copy(pos_hbm.at[pl.ds(sid * pr, pr)],
                        pos_sh.at[pl.ds(sid * pr, pr)])
        pltpu.sync_copy(seg_hbm.at[pl.ds(sid * sr, sr)],
                        seg_sh.at[pl.ds(sid * sr, sr)])

        prem = max_len - 16 * pr
        srem = max_seg - 16 * sr
        if prem:
            @pl.when(sid == 0)
            def _():
                pltpu.sync_copy(pos_hbm.at[pl.ds(16 * pr, prem)],
                                pos_sh.at[pl.ds(16 * pr, prem)])
        if srem:
            @pl.when(sid == 1)
            def _():
                pltpu.sync_copy(seg_hbm.at[pl.ds(16 * sr, srem)],
                                seg_sh.at[pl.ds(16 * sr, srem)])

        plsc.subcore_barrier()
        c_pv.wait()
        c_sv.wait()

        def load_ix(c, q):
            pltpu.async_copy(xi_hbm.at[pl.ds(base + c * W, W)],
                             ixbuf[q], six[q])

        def issue_w(q):
            pltpu.async_copy(word_hbm.at[ixbuf[q]], wbuf[q], sw[q])

        def issue_p(c, q):
            pltpu.async_copy(pos_sh.at[pv.at[pl.ds(c * W, W)]],
                             psbuf[q], sp[q])

        def issue_s(c, q):
            pltpu.async_copy(seg_sh.at[sv.at[pl.ds(c * W, W)]],
                             psbuf[q], ss[q], add=True)

        def wait_rows(sem, buf):
            pltpu.make_async_copy(o_hbm.at[pl.ds(0, W)], buf, sem).wait()

        def wait_ix(q):
            pltpu.make_async_copy(xi_hbm.at[pl.ds(0, W)], ixbuf[q],
                                  six[q]).wait()

        issue_p(0, 0)
        wait_rows(sp[0], psbuf[0])
        issue_s(0, 0)

        def body(c, q):
            @pl.when(c + 1 < n_chunks)
            def _():
                wait_ix(1 - q)
                issue_w(1 - q)

            wait_rows(sw[q], wbuf[q])

            @pl.when(c + 2 < n_chunks)
            def _():
                load_ix(c + 2, q)

            wait_rows(ss[q], psbuf[q])

            @pl.when(c + 1 < n_chunks)
            def _():
                @pl.when(c >= 1)
                def _():
                    pltpu.make_async_copy(
                        psbuf[1 - q], o_hbm.at[pl.ds(base, W)],
                        so[1 - q]).wait()

                issue_p(c + 1, 1 - q)

            @plsc.parallel_loop(0, W, step=1, unroll=4)
            def _(r):
                for col in range(0, D, 16):
                    sl = (r, pl.ds(col, 16))
                    psbuf[q][sl] = wbuf[q][sl] * SCALE + psbuf[q][sl]

            pltpu.async_copy(psbuf[q], o_hbm.at[pl.ds(base + c * W, W)],
                             so[q])

            @pl.when(c + 1 < n_chunks)
            def _():
                wait_rows(sp[1 - q], psbuf[1 - q])
                issue_s(c + 1, 1 - q)

        @pl.loop(0, n_chunks, step=2)
        def _(c):
            body(c, 0)
            body(c + 1, 1)

        pltpu.make_async_copy(psbuf[0], o_hbm.at[pl.ds(base, W)], so[0]).wait()
        pltpu.make_async_copy(psbuf[1], o_hbm.at[pl.ds(base, W)], so[1]).wait()

    return kern(word_emb, pos_emb, seg_emb, xi, pi, si)


def kernel(x, pos, seg, word_emb, pos_emb, seg_emb):
    b, l = x.shape
    n_tok = b * l
    xi = x.reshape(n_tok).astype(jnp.int32)
    pi = pos.reshape(n_tok).astype(jnp.int32)
    si = seg.reshape(n_tok).astype(jnp.int32)
    out = _seg_embedding_sc(xi, pi, si, word_emb, pos_emb, seg_emb)
    return out.reshape(b, l, D)
